# baseline (speedup 1.0000x reference)
import functools

import jax
import jax.numpy as jnp
from jax import lax
from jax.experimental import pallas as pl
from jax.experimental.pallas import tpu as pltpu
from jax.experimental.pallas import tpu_sc as plsc

EPS = 1e-12
LANES = 16


_GATHER_DNUMS = lax.GatherDimensionNumbers(
    offset_dims=(), collapsed_slice_dims=(0,), start_index_map=(0,))


def _lane_gather(x, idx):
    return lax.gather(x, idx[:, None], _GATHER_DNUMS, slice_sizes=(1,),
                      mode=lax.GatherScatterMode.PROMISE_IN_BOUNDS)


def _lane_allsum(x):
    iota = lax.iota(jnp.int32, LANES)
    for k in (8, 4, 2, 1):
        idx = jnp.bitwise_and(iota + k, LANES - 1)
        x = x + _lane_gather(x, idx)
    return x


def _rsqrt_vec(x):
    i = lax.bitcast_convert_type(x, jnp.int32)
    i = jnp.int32(0x5F3759DF) - lax.shift_right_logical(i, 1)
    y = lax.bitcast_convert_type(i, jnp.float32)
    for _ in range(3):
        y = y * (1.5 - 0.5 * x * y * y)
    return y


@functools.lru_cache(maxsize=None)
def _build(T, S, D, C):
    info = plsc.get_sparse_core_info()
    NC, NS = info.num_cores, info.num_subcores
    NW = NC * NS
    per_w = T // NW
    n_chunks = per_w // C
    NV = D // LANES

    mesh = plsc.VectorSubcoreMesh(core_axis_name="c", subcore_axis_name="s")

    @functools.partial(
        pl.kernel,
        mesh=mesh,
        out_type=jax.ShapeDtypeStruct((T, D), jnp.float32),
        scratch_types=[
            pltpu.VMEM((C,), jnp.int32),
            pltpu.VMEM((C, D), jnp.float32),
            pltpu.VMEM((C, D), jnp.float32),
            pltpu.VMEM((D,), jnp.float32),
            pltpu.VMEM((D,), jnp.float32),
            pltpu.SemaphoreType.DMA,
            pltpu.SemaphoreType.DMA,
        ],
    )
    def embed_ln(ids_hbm, wt_hbm, pt_hbm, g_hbm, b_hbm, out_hbm,
                 idx_v, xbuf, pbuf, g_v, b_v, sem_w, sem_p):
        wid = lax.axis_index("s") * NC + lax.axis_index("c")
        base = wid * per_w
        pltpu.sync_copy(g_hbm, g_v)
        pltpu.sync_copy(b_hbm, b_v)

        def chunk_body(c, _):
            fb = base + c * C
            s0 = lax.rem(fb, S)
            pltpu.sync_copy(ids_hbm.at[pl.ds(fb, C)], idx_v)
            cp_w = pltpu.async_copy(wt_hbm.at[idx_v], xbuf, sem_w)
            cp_p = pltpu.async_copy(pt_hbm.at[pl.ds(s0, C)], pbuf, sem_p)
            cp_w.wait()
            cp_p.wait()

            def tok_body(t, _):
                zero = jnp.zeros((LANES,), jnp.float32)
                acc = [zero, zero, zero, zero]
                acc2 = [zero, zero, zero, zero]
                for j in range(NV):
                    x = xbuf[t, pl.ds(j * LANES, LANES)] + \
                        pbuf[t, pl.ds(j * LANES, LANES)]
                    xbuf[t, pl.ds(j * LANES, LANES)] = x
                    acc[j % 4] = acc[j % 4] + x
                    acc2[j % 4] = acc2[j % 4] + x * x
                s1 = _lane_allsum((acc[0] + acc[1]) + (acc[2] + acc[3]))
                s2 = _lane_allsum((acc2[0] + acc2[1]) + (acc2[2] + acc2[3]))
                mean_v = s1 * (1.0 / D)
                var_v = s2 * (1.0 / D) - mean_v * mean_v
                r_v = _rsqrt_vec(var_v + EPS)
                for j in range(NV):
                    x = xbuf[t, pl.ds(j * LANES, LANES)]
                    y = (x - mean_v) * r_v * g_v[pl.ds(j * LANES, LANES)] \
                        + b_v[pl.ds(j * LANES, LANES)]
                    xbuf[t, pl.ds(j * LANES, LANES)] = y
                return 0

            lax.fori_loop(0, C, tok_body, 0)
            pltpu.sync_copy(xbuf, out_hbm.at[pl.ds(fb, C)])
            return 0

        lax.fori_loop(0, n_chunks, chunk_body, 0)

    return embed_ln


def kernel(input_ids, word_table, pos_table, gamma, beta):
    B, S = input_ids.shape
    V, D = word_table.shape
    T = B * S
    ids_flat = input_ids.reshape(T).astype(jnp.int32)
    fn = _build(T, S, D, 64)
    out = fn(ids_flat, word_table, pos_table, gamma, beta)
    return out.reshape(B, S, D)
